# Initial kernel scaffold; baseline (speedup 1.0000x reference)
import jax
import jax.numpy as jnp
from jax.experimental import pallas as pl


def kernel(x, w1, b1, w3a, b3a, w3b, b3b, w5a, b5a, w5b, b5b, wp, bp):
    raise NotImplementedError("write your pallas kernel here")


# speedup vs baseline: 2.4254x; 2.4254x over previous
import jax
import jax.numpy as jnp
from jax import lax
from jax.experimental import pallas as pl
from jax.experimental.pallas import tpu as pltpu


def _make_kernel(H, W, Cin, C1, C3, Ctot):
    P = H * W

    def body(x_ref, wb_ref, wm_ref, bb_ref, bo_ref, o_ref):
        xb = x_ref[0].astype(jnp.bfloat16)

        lane = lax.broadcasted_iota(jnp.int32, (1, P), 1)
        wi = lane % W
        hi = lane // W

        def shift_w(a, d, fill):
            if d == 0:
                return a
            m = (wi + d >= 0) & (wi + d < W)
            return jnp.where(m, pltpu.roll(a, (-d) % P, axis=1), fill)

        def shift_h(a, d, fill):
            if d == 0:
                return a
            m = (hi + d >= 0) & (hi + d < H)
            return jnp.where(m, pltpu.roll(a, (-d * W) % P, axis=1), fill)

        t = jnp.dot(wb_ref[...], xb, preferred_element_type=jnp.float32)
        t = (t + bb_ref[...]).astype(jnp.bfloat16)
        t3 = t[:C1]
        t5 = t[C1:]

        zero = jnp.zeros((), jnp.bfloat16)
        t3dw = jnp.concatenate([shift_w(t3, d, zero) for d in (-1, 0, 1)],
                               axis=0)
        cat3 = jnp.concatenate([shift_h(t3dw, d, zero) for d in (-1, 0, 1)],
                               axis=0)
        t5dw = jnp.concatenate([shift_w(t5, d, zero)
                                for d in (-2, -1, 0, 1, 2)], axis=0)
        cat5 = jnp.concatenate([shift_h(t5dw, d, zero)
                                for d in (-2, -1, 0, 1, 2)], axis=0)

        neg = jnp.array(-jnp.inf, jnp.bfloat16)
        m1 = jnp.maximum(jnp.maximum(shift_w(xb, -1, neg),
                                     shift_w(xb, 1, neg)), xb)
        mp = jnp.maximum(jnp.maximum(shift_h(m1, -1, neg),
                                     shift_h(m1, 1, neg)), m1)

        cat = jnp.concatenate([xb, mp, cat3, cat5], axis=0)
        acc = jnp.dot(wm_ref[...], cat, preferred_element_type=jnp.float32)
        o_ref[0] = acc + bo_ref[...]

    return body


def kernel(x, w1, b1, w3a, b3a, w3b, b3b, w5a, b5a, w5b, b5b, wp, bp):
    N, Cin, H, W = x.shape
    C0 = w1.shape[1]
    C1, C2 = w3b.shape[2], w3b.shape[3]
    C3, C4 = w5b.shape[2], w5b.shape[3]
    C5 = wp.shape[1]
    Ctot = C0 + C2 + C4 + C5
    P = H * W
    K = 2 * Cin + 9 * C1 + 25 * C3

    f32, bf16 = jnp.float32, jnp.bfloat16

    wb = jnp.concatenate([w3a.T, w5a.T], axis=0).astype(bf16)
    bb = jnp.concatenate([b3a.reshape(-1), b5a.reshape(-1)])[:, None].astype(f32)

    wm = jnp.zeros((Ctot, K), f32)
    wm = wm.at[:C0, :Cin].set(w1.T)
    wm = wm.at[C0 + C2 + C4:, Cin:2 * Cin].set(wp.T)
    off = 2 * Cin
    for dh in range(3):
        for dw in range(3):
            wm = wm.at[C0:C0 + C2,
                       off + C1 * (3 * dh + dw):
                       off + C1 * (3 * dh + dw + 1)].set(w3b[dh, dw].T)
    off = 2 * Cin + 9 * C1
    for dh in range(5):
        for dw in range(5):
            wm = wm.at[C0 + C2:C0 + C2 + C4,
                       off + C3 * (5 * dh + dw):
                       off + C3 * (5 * dh + dw + 1)].set(w5b[dh, dw].T)
    wm = wm.astype(bf16)

    bo = jnp.concatenate([b1.reshape(-1), b3b.reshape(-1),
                          b5b.reshape(-1), bp.reshape(-1)])[:, None].astype(f32)

    x_flat = x.reshape(N, Cin, P)

    def full_spec(a):
        nd = a.ndim
        return pl.BlockSpec(a.shape, lambda n, _nd=nd: (0,) * _nd)

    out = pl.pallas_call(
        _make_kernel(H, W, Cin, C1, C3, Ctot),
        out_shape=jax.ShapeDtypeStruct((N, Ctot, P), f32),
        grid=(N,),
        in_specs=[pl.BlockSpec((1, Cin, P), lambda n: (n, 0, 0)),
                  full_spec(wb), full_spec(wm),
                  full_spec(bb), full_spec(bo)],
        out_specs=pl.BlockSpec((1, Ctot, P), lambda n: (n, 0, 0)),
        compiler_params=pltpu.CompilerParams(
            dimension_semantics=("parallel",)),
    )(x_flat, wb, wm, bb, bo)

    return out.reshape(N, Ctot, H, W)


# speedup vs baseline: 2.8654x; 1.1814x over previous
import jax
import jax.numpy as jnp
from jax import lax
from jax.experimental import pallas as pl
from jax.experimental.pallas import tpu as pltpu


def _make_kernel(H, W, Cin, C1, C3, Ctot, B):
    P = H * W

    def body(x_ref, wb_ref, wm_ref, bb_ref, bo_ref, o_ref):
        lane = lax.broadcasted_iota(jnp.int32, (1, P), 1)
        wi = lane % W
        hi = lane // W

        def shift_w(a, d, fill):
            if d == 0:
                return a
            m = (wi + d >= 0) & (wi + d < W)
            return jnp.where(m, pltpu.roll(a, (-d) % P, axis=1), fill)

        def shift_h(a, d, fill):
            if d == 0:
                return a
            m = (hi + d >= 0) & (hi + d < H)
            return jnp.where(m, pltpu.roll(a, (-d * W) % P, axis=1), fill)

        for b in range(B):
            xb = x_ref[b].astype(jnp.bfloat16)

            t = jnp.dot(wb_ref[...], xb, preferred_element_type=jnp.float32)
            t = (t + bb_ref[...]).astype(jnp.bfloat16)
            t3 = t[:C1]
            t5 = t[C1:]

            zero = jnp.zeros((), jnp.bfloat16)
            t3dw = jnp.concatenate([shift_w(t3, d, zero) for d in (-1, 0, 1)],
                                   axis=0)
            cat3 = jnp.concatenate([shift_h(t3dw, d, zero) for d in (-1, 0, 1)],
                                   axis=0)
            t5dw = jnp.concatenate([shift_w(t5, d, zero)
                                    for d in (-2, -1, 0, 1, 2)], axis=0)
            cat5 = jnp.concatenate([shift_h(t5dw, d, zero)
                                    for d in (-2, -1, 0, 1, 2)],
                                   axis=0)

            neg = jnp.array(-jnp.inf, jnp.bfloat16)
            m1 = jnp.maximum(jnp.maximum(shift_w(xb, -1, neg),
                                         shift_w(xb, 1, neg)), xb)
            mp = jnp.maximum(jnp.maximum(shift_h(m1, -1, neg),
                                         shift_h(m1, 1, neg)), m1)

            cat = jnp.concatenate([xb, mp, cat3, cat5], axis=0)
            acc = jnp.dot(wm_ref[...], cat,
                          preferred_element_type=jnp.float32)
            o_ref[b] = acc + bo_ref[...]

    return body


def kernel(x, w1, b1, w3a, b3a, w3b, b3b, w5a, b5a, w5b, b5b, wp, bp):
    N, Cin, H, W = x.shape
    C0 = w1.shape[1]
    C1, C2 = w3b.shape[2], w3b.shape[3]
    C3, C4 = w5b.shape[2], w5b.shape[3]
    C5 = wp.shape[1]
    Ctot = C0 + C2 + C4 + C5
    P = H * W
    K = 2 * Cin + 9 * C1 + 25 * C3

    f32, bf16 = jnp.float32, jnp.bfloat16

    wb = jnp.concatenate([w3a.T, w5a.T], axis=0).astype(bf16)
    bb = jnp.concatenate([b3a.reshape(-1), b5a.reshape(-1)])[:, None].astype(f32)

    wm = jnp.zeros((Ctot, K), f32)
    wm = wm.at[:C0, :Cin].set(w1.T)
    wm = wm.at[C0 + C2 + C4:, Cin:2 * Cin].set(wp.T)
    off = 2 * Cin
    for dh in range(3):
        for dw in range(3):
            wm = wm.at[C0:C0 + C2,
                       off + C1 * (3 * dh + dw):
                       off + C1 * (3 * dh + dw + 1)].set(w3b[dh, dw].T)
    off = 2 * Cin + 9 * C1
    for dh in range(5):
        for dw in range(5):
            wm = wm.at[C0 + C2:C0 + C2 + C4,
                       off + C3 * (5 * dh + dw):
                       off + C3 * (5 * dh + dw + 1)].set(w5b[dh, dw].T)
    wm = wm.astype(bf16)

    bo = jnp.concatenate([b1.reshape(-1), b3b.reshape(-1),
                          b5b.reshape(-1), bp.reshape(-1)])[:, None].astype(f32)

    x_flat = x.reshape(N, Cin, P)

    B = 8
    while N % B:
        B //= 2

    def full_spec(a):
        nd = a.ndim
        return pl.BlockSpec(a.shape, lambda n, _nd=nd: (0,) * _nd)

    out = pl.pallas_call(
        _make_kernel(H, W, Cin, C1, C3, Ctot, B),
        out_shape=jax.ShapeDtypeStruct((N, Ctot, P), f32),
        grid=(N // B,),
        in_specs=[pl.BlockSpec((B, Cin, P), lambda n: (n, 0, 0)),
                  full_spec(wb), full_spec(wm),
                  full_spec(bb), full_spec(bo)],
        out_specs=pl.BlockSpec((B, Ctot, P), lambda n: (n, 0, 0)),
        compiler_params=pltpu.CompilerParams(
            dimension_semantics=("parallel",)),
    )(x_flat, wb, wm, bb, bo)

    return out.reshape(N, Ctot, H, W)
